# Initial kernel scaffold; baseline (speedup 1.0000x reference)
import jax
import jax.numpy as jnp
from jax.experimental import pallas as pl


def kernel(x, edge_index, batch, edge_attr, ae_w, ae_b, be_w, be_b, eps, c_w1, c_b1, c_g1, c_bt1, c_w2, c_b2, bn_g, bn_b, l_wih, l_whh, l_bih, l_bhh, h_w1, h_b1, h_g1, h_bt1, h_w2, h_b2, h_g2, h_bt2, h_w3, h_b3):
    raise NotImplementedError("write your pallas kernel here")


# speedup vs baseline: 1.7724x; 1.7724x over previous
import functools
import math

import jax
import jax.numpy as jnp
from jax import lax
from jax.experimental import pallas as pl
from jax.experimental.pallas import tpu as pltpu
from jax.experimental.pallas import tpu_sc as plsc

N = 10000
E = 160000
DN = 128
DE = 16
HID = 256
QW = 64
NQ = HID // QW
NB = 256
LAYERS = 5

NC = 2
NS = 16
ES = E // NS
C = 80
NCH = ES // C
NR = 624
NTAIL = N - NR * NS

_BN_INV = 1.0 / math.sqrt(1.0 + 1e-5)


def _sc_quarter(s, h_ref, ea_ref, zeros, out_ref,
                src_v, dst_v, rows_v, ea_v, aggr, sem):
    pltpu.sync_copy(zeros.at[pl.ds(s * NR, NR)], aggr.at[pl.ds(s * NR, NR)])

    @pl.when(s == 0)
    def _():
        pltpu.sync_copy(zeros.at[pl.ds(NR * NS, NTAIL)],
                        aggr.at[pl.ds(NR * NS, NTAIL)])

    plsc.subcore_barrier()
    ebase = s * ES

    def chunk(j, carry):
        pltpu.async_copy(h_ref.at[src_v.at[j]], rows_v, sem).wait()
        pltpu.sync_copy(ea_ref.at[pl.ds(ebase + j * C, C)], ea_v)

        def edge(e, c2):
            for k in range(QW // 16):
                sl = pl.ds(k * 16, 16)
                rows_v[e, sl] = jnp.maximum(rows_v[e, sl] + ea_v[e, sl], 0.0)
            return c2

        lax.fori_loop(0, C, edge, 0)
        pltpu.sync_copy(rows_v, aggr.at[dst_v.at[j]], add=True)
        return carry

    lax.fori_loop(0, NCH, chunk, 0)
    plsc.subcore_barrier()
    pltpu.sync_copy(aggr.at[pl.ds(s * NR, NR)], out_ref.at[pl.ds(s * NR, NR)])

    @pl.when(s == 0)
    def _():
        pltpu.sync_copy(aggr.at[pl.ds(NR * NS, NTAIL)],
                        out_ref.at[pl.ds(NR * NS, NTAIL)])

    plsc.subcore_barrier()


@functools.partial(
    pl.kernel,
    out_type=[jax.ShapeDtypeStruct((N, QW), jnp.float32)] * NQ,
    mesh=plsc.VectorSubcoreMesh(core_axis_name="c", subcore_axis_name="s"),
    compiler_params=pltpu.CompilerParams(use_tc_tiling_on_sc=False),
    scratch_types=[
        pltpu.VMEM((NCH, C), jnp.int32),
        pltpu.VMEM((NCH, C), jnp.int32),
        pltpu.VMEM((C, QW), jnp.float32),
        pltpu.VMEM((C, QW), jnp.float32),
        pltpu.VMEM_SHARED((N, QW), jnp.float32),
        pltpu.SemaphoreType.DMA,
    ],
)
def _sc_msg(h0, h1, h2, h3, ea0, ea1, ea2, ea3, src3d, dst3d, zeros,
            out0, out1, out2, out3,
            src_v, dst_v, rows_v, ea_v, aggr, sem):
    c = lax.axis_index("c")
    s = lax.axis_index("s")
    pltpu.sync_copy(src3d.at[s], src_v)
    pltpu.sync_copy(dst3d.at[s], dst_v)

    @pl.when(c == 0)
    def _():
        _sc_quarter(s, h0, ea0, zeros, out0, src_v, dst_v, rows_v, ea_v,
                    aggr, sem)
        _sc_quarter(s, h1, ea1, zeros, out1, src_v, dst_v, rows_v, ea_v,
                    aggr, sem)

    @pl.when(c == 1)
    def _():
        _sc_quarter(s, h2, ea2, zeros, out2, src_v, dst_v, rows_v, ea_v,
                    aggr, sem)
        _sc_quarter(s, h3, ea3, zeros, out3, src_v, dst_v, rows_v, ea_v,
                    aggr, sem)


def _embed_body(x_ref, w_ref, b_ref, o0_ref, o1_ref, o2_ref, o3_ref):
    r = jnp.dot(x_ref[...], w_ref[...],
                preferred_element_type=jnp.float32) + b_ref[...]
    o0_ref[...] = r[:, 0 * QW:1 * QW]
    o1_ref[...] = r[:, 1 * QW:2 * QW]
    o2_ref[...] = r[:, 2 * QW:3 * QW]
    o3_ref[...] = r[:, 3 * QW:4 * QW]


def _embed(x, w, b, block):
    rows, din = x.shape
    grid = rows // block
    return pl.pallas_call(
        _embed_body,
        grid=(grid,),
        in_specs=[
            pl.BlockSpec((block, din), lambda i: (i, 0)),
            pl.BlockSpec((din, HID), lambda i: (0, 0)),
            pl.BlockSpec((1, HID), lambda i: (0, 0)),
        ],
        out_specs=[pl.BlockSpec((block, QW), lambda i: (i, 0))] * NQ,
        out_shape=[jax.ShapeDtypeStruct((rows, QW), jnp.float32)] * NQ,
    )(x, w, b.reshape(1, HID))


def _mlp_body(ep_ref, h0_ref, h1_ref, h2_ref, h3_ref,
              a0_ref, a1_ref, a2_ref, a3_ref, w1_ref, b1_ref,
              g1_ref, t1_ref, w2_ref, b2_ref, g2_ref, t2_ref,
              o0_ref, o1_ref, o2_ref, o3_ref):
    ep = ep_ref[...]
    hs = [h0_ref[...], h1_ref[...], h2_ref[...], h3_ref[...]]
    As = [a0_ref[...], a1_ref[...], a2_ref[...], a3_ref[...]]
    u = b1_ref[...]
    for q in range(NQ):
        zq = ep * hs[q] + As[q]
        u = u + jnp.dot(zq, w1_ref[q * QW:(q + 1) * QW, :],
                        preferred_element_type=jnp.float32)
    u = jnp.maximum(u * g1_ref[...] + t1_ref[...], 0.0)
    v = jnp.dot(u, w2_ref[...], preferred_element_type=jnp.float32) + b2_ref[...]
    v = jnp.maximum(v * g2_ref[...] + t2_ref[...], 0.0)
    o0_ref[...] = v[:, 0 * QW:1 * QW] + hs[0]
    o1_ref[...] = v[:, 1 * QW:2 * QW] + hs[1]
    o2_ref[...] = v[:, 2 * QW:3 * QW] + hs[2]
    o3_ref[...] = v[:, 3 * QW:4 * QW] + hs[3]


def _mlp(ep, hq, aq, w1, b1, g1, t1, w2, b2, g2, t2, block=2000):
    grid = N // block
    qspec = pl.BlockSpec((block, QW), lambda i: (i, 0))
    return pl.pallas_call(
        _mlp_body,
        grid=(grid,),
        in_specs=[
            pl.BlockSpec((1, QW), lambda i: (0, 0)),
            qspec, qspec, qspec, qspec,
            qspec, qspec, qspec, qspec,
            pl.BlockSpec((HID, 2 * HID), lambda i: (0, 0)),
            pl.BlockSpec((1, 2 * HID), lambda i: (0, 0)),
            pl.BlockSpec((1, 2 * HID), lambda i: (0, 0)),
            pl.BlockSpec((1, 2 * HID), lambda i: (0, 0)),
            pl.BlockSpec((2 * HID, HID), lambda i: (0, 0)),
            pl.BlockSpec((1, HID), lambda i: (0, 0)),
            pl.BlockSpec((1, HID), lambda i: (0, 0)),
            pl.BlockSpec((1, HID), lambda i: (0, 0)),
        ],
        out_specs=[pl.BlockSpec((block, QW), lambda i: (i, 0))] * NQ,
        out_shape=[jax.ShapeDtypeStruct((N, QW), jnp.float32)] * NQ,
    )(ep, *hq, *aq, w1, b1.reshape(1, -1), g1.reshape(1, -1),
      t1.reshape(1, -1), w2, b2.reshape(1, -1), g2.reshape(1, -1),
      t2.reshape(1, -1))


RNR = 624
RCH = 13
RC = RNR // RCH
RTAIL = N - RNR * NS


def _s2s_step_body(h0_ref, h1_ref, h2_ref, h3_ref, batch_ref, qs_ref,
                   hh_ref, cc_ref, wih_ref, whh_ref, bi_ref, bh_ref,
                   a_ref, q_ref, hho_ref, cco_ref):
    hqs = [h0_ref[...], h1_ref[...], h2_ref[...], h3_ref[...]]
    seg = lax.broadcasted_iota(jnp.int32, (N, NB), 1)
    of = (batch_ref[...] == seg).astype(jnp.float32)

    gates = (jnp.dot(qs_ref[...], wih_ref[...],
                     preferred_element_type=jnp.float32) + bi_ref[...]
             + jnp.dot(hh_ref[...], whh_ref[...],
                       preferred_element_type=jnp.float32) + bh_ref[...])
    ig = jax.nn.sigmoid(gates[:, :HID])
    fg = jax.nn.sigmoid(gates[:, HID:2 * HID])
    gg = jnp.tanh(gates[:, 2 * HID:3 * HID])
    og = jax.nn.sigmoid(gates[:, 3 * HID:])
    cc = fg * cc_ref[...] + ig * gg
    hh = og * jnp.tanh(cc)
    q = hh

    e = jnp.zeros((N, 1), jnp.float32)
    for qi in range(NQ):
        qn = jnp.dot(of, q[:, qi * QW:(qi + 1) * QW],
                     preferred_element_type=jnp.float32)
        e = e + jnp.sum(hqs[qi] * qn, axis=1, keepdims=True)
    emin = jnp.min(e)
    es = e - emin + 1.0
    ms = jnp.max(of * es, axis=0, keepdims=True)
    m_safe = jnp.where(ms == 0.0, 0.0, ms + (emin - 1.0))
    mn = jnp.sum(of * m_safe, axis=1, keepdims=True)
    ex = jnp.exp(e - mn)
    sseg = jnp.sum(of * ex, axis=0, keepdims=True)
    sn = jnp.sum(of * sseg, axis=1, keepdims=True)
    a_ref[...] = ex / (sn + 1e-16)
    q_ref[...] = q
    hho_ref[...] = hh
    cco_ref[...] = cc


def _s2s_step(hq, batch2d, qs, hh, cc, wih_t, whh_t, bi, bh):
    return pl.pallas_call(
        _s2s_step_body,
        out_shape=[jax.ShapeDtypeStruct((N, 1), jnp.float32),
                   jax.ShapeDtypeStruct((NB, HID), jnp.float32),
                   jax.ShapeDtypeStruct((NB, HID), jnp.float32),
                   jax.ShapeDtypeStruct((NB, HID), jnp.float32)],
    )(*hq, batch2d, qs, hh, cc, wih_t, whh_t, bi, bh)


def _sc_r_half(s, hA, hB, a3d, a_t, b3d, b_t, zeros_r, out_ref,
               idx_v, av_v, idxt_v, at_v, h0_buf, h1_buf, ha_buf, racc, sem):
    pltpu.sync_copy(zeros_r.at[pl.ds(s * 16, 16)], racc.at[pl.ds(s * 16, 16)])
    pltpu.sync_copy(b3d.at[s], idx_v)
    pltpu.sync_copy(a3d.at[s], av_v)
    plsc.subcore_barrier()
    def chunk(j, carry):
        nbase = s * RNR + j * RC
        pltpu.sync_copy(hA.at[pl.ds(nbase, RC)], h0_buf)
        pltpu.sync_copy(hB.at[pl.ds(nbase, RC)], h1_buf)

        for g in range(RC // 16):
            av = av_v[j, pl.ds(g * 16, 16)]
            for lane in range(16):
                aa = av[lane]
                e = g * 16 + lane
                for k in range(QW // 16):
                    sl = pl.ds(k * 16, 16)
                    sr = pl.ds(QW + k * 16, 16)
                    ha_buf[e, sl] = h0_buf[e, sl] * aa
                    ha_buf[e, sr] = h1_buf[e, sl] * aa
        pltpu.sync_copy(ha_buf, racc.at[idx_v.at[j]], add=True)
        return carry

    lax.fori_loop(0, RCH, chunk, 0)

    @pl.when(s == 0)
    def _():
        pltpu.sync_copy(b_t, idxt_v)
        pltpu.sync_copy(a_t, at_v)
        pltpu.sync_copy(hA.at[pl.ds(RNR * NS, RTAIL)],
                        h0_buf.at[pl.ds(0, RTAIL)])
        pltpu.sync_copy(hB.at[pl.ds(RNR * NS, RTAIL)],
                        h1_buf.at[pl.ds(0, RTAIL)])

        av = at_v[0, pl.ds(0, 16)]
        for lane in range(RTAIL):
            aa = av[lane]
            for k in range(QW // 16):
                sl = pl.ds(k * 16, 16)
                sr = pl.ds(QW + k * 16, 16)
                ha_buf[lane, sl] = h0_buf[lane, sl] * aa
                ha_buf[lane, sr] = h1_buf[lane, sl] * aa
        pltpu.sync_copy(ha_buf.at[pl.ds(0, RTAIL)],
                        racc.at[idxt_v.at[0]], add=True)

    plsc.subcore_barrier()
    pltpu.sync_copy(racc.at[pl.ds(s * 16, 16)], out_ref.at[pl.ds(s * 16, 16)])
    plsc.subcore_barrier()


@functools.partial(
    pl.kernel,
    out_type=[jax.ShapeDtypeStruct((NB, 2 * QW), jnp.float32)] * 2,
    mesh=plsc.VectorSubcoreMesh(core_axis_name="c", subcore_axis_name="s"),
    compiler_params=pltpu.CompilerParams(use_tc_tiling_on_sc=False),
    scratch_types=[
        pltpu.VMEM((RCH, RC), jnp.int32),
        pltpu.VMEM((RCH, RC), jnp.float32),
        pltpu.VMEM((1, RTAIL), jnp.int32),
        pltpu.VMEM((1, RTAIL), jnp.float32),
        pltpu.VMEM((RC, QW), jnp.float32),
        pltpu.VMEM((RC, QW), jnp.float32),
        pltpu.VMEM((RC, 2 * QW), jnp.float32),
        pltpu.VMEM_SHARED((NB, 2 * QW), jnp.float32),
        pltpu.SemaphoreType.DMA,
    ],
)
def _sc_readout(h0, h1, h2, h3, a3d, a_t, b3d, b_t, zeros_r, out0, out1,
                idx_v, av_v, idxt_v, at_v, h0_buf, h1_buf, ha_buf, racc, sem):
    c = lax.axis_index("c")
    s = lax.axis_index("s")

    @pl.when(c == 0)
    def _():
        _sc_r_half(s, h0, h1, a3d, a_t, b3d, b_t, zeros_r, out0,
                   idx_v, av_v, idxt_v, at_v, h0_buf, h1_buf, ha_buf,
                   racc, sem)

    @pl.when(c == 1)
    def _():
        _sc_r_half(s, h2, h3, a3d, a_t, b3d, b_t, zeros_r, out1,
                   idx_v, av_v, idxt_v, at_v, h0_buf, h1_buf, ha_buf,
                   racc, sem)


def _cls_body(qs_ref, w1_ref, b1_ref, g1_ref, t1_ref,
              w2_ref, b2_ref, g2_ref, t2_ref, w3_ref, b3_ref, out_ref):
    z = (jnp.dot(qs_ref[...], w1_ref[...], preferred_element_type=jnp.float32)
         + b1_ref[...])
    z = jnp.maximum(z * g1_ref[...] + t1_ref[...], 0.0)
    z = jnp.dot(z, w2_ref[...], preferred_element_type=jnp.float32) + b2_ref[...]
    z = jnp.maximum(z * g2_ref[...] + t2_ref[...], 0.0)
    out_ref[...] = (jnp.dot(z, w3_ref[...], preferred_element_type=jnp.float32)
                    + b3_ref[...])


def _classifier(qs, w1, b1, g1, t1, w2, b2, g2, t2, w3, b3):
    return pl.pallas_call(
        _cls_body,
        out_shape=jax.ShapeDtypeStruct((NB, 1), jnp.float32),
    )(qs, w1, b1.reshape(1, HID), g1.reshape(1, HID), t1.reshape(1, HID),
      w2, b2.reshape(1, HID // 2), g2.reshape(1, HID // 2),
      t2.reshape(1, HID // 2), w3, b3.reshape(1, 1))


def kernel(x, edge_index, batch, edge_attr, ae_w, ae_b, be_w, be_b, eps,
           c_w1, c_b1, c_g1, c_bt1, c_w2, c_b2, bn_g, bn_b,
           l_wih, l_whh, l_bih, l_bhh,
           h_w1, h_b1, h_g1, h_bt1, h_w2, h_b2, h_g2, h_bt2, h_w3, h_b3):
    hq = _embed(x, ae_w, ae_b, block=2000)
    eaq = _embed(edge_attr, be_w, be_b, block=2000)
    src3d = edge_index[0].reshape(NS, NCH, C)
    dst3d = edge_index[1].reshape(NS, NCH, C)
    zeros = jnp.zeros((N, QW), jnp.float32)

    for i in range(LAYERS):
        aq = _sc_msg(*hq, *eaq, src3d, dst3d, zeros)
        ep = jnp.full((1, QW), 1.0 + eps[i], jnp.float32)
        hq = _mlp(ep, hq, aq,
                  c_w1[i], c_b1[i], c_g1[i] * _BN_INV, c_bt1[i],
                  c_w2[i], c_b2[i], bn_g[i] * _BN_INV, bn_b[i])

    batch2d = batch.reshape(N, 1)
    b3d = batch[:RNR * NS].reshape(NS, RCH, RC)
    b_t = batch[RNR * NS:].reshape(1, RTAIL)
    zeros_r = jnp.zeros((NB, 2 * QW), jnp.float32)
    wih_t = l_wih.T
    whh_t = l_whh.T
    bi = l_bih.reshape(1, 4 * HID)
    bh = l_bhh.reshape(1, 4 * HID)

    qs = jnp.zeros((NB, 2 * HID), jnp.float32)
    hh = jnp.zeros((NB, HID), jnp.float32)
    cc = jnp.zeros((NB, HID), jnp.float32)
    for _ in range(3):
        a, q, hh, cc = _s2s_step(hq, batch2d, qs, hh, cc,
                                 wih_t, whh_t, bi, bh)
        a1 = a[:, 0]
        a3d = a1[:RNR * NS].reshape(NS, RCH, RC)
        a_t = a1[RNR * NS:].reshape(1, RTAIL)
        r0, r1 = _sc_readout(*hq, a3d, a_t, b3d, b_t, zeros_r)
        qs = jnp.concatenate([q, r0, r1], axis=1)

    out = _classifier(qs, h_w1, h_b1, h_g1 * _BN_INV, h_bt1,
                      h_w2, h_b2, h_g2 * _BN_INV, h_bt2, h_w3, h_b3)
    return out[:, 0]
